# baseline (speedup 1.0000x reference)
import jax
import jax.numpy as jnp
from jax.experimental import pallas as pl
from jax.experimental.pallas import tpu as pltpu

_PC_X0 = -54.0
_PC_Y0 = -54.0
_VX = 0.075 * 4.0
_VY = 0.075 * 4.0
_LOG_EPS64 = -36.04365338911715
_NEG = -3.0e38

_H = 360
_W = 360
_SLAB = 40


def _radius(height, width, min_overlap=0.5):
    b1 = height + width
    c1 = width * height * (1 - min_overlap) / (1 + min_overlap)
    sq1 = jnp.sqrt(jnp.maximum(b1 * b1 - 4 * c1, 0.0))
    r1 = (b1 + sq1) / 2
    b2 = 2 * (height + width)
    c2 = (1 - min_overlap) * width * height
    sq2 = jnp.sqrt(jnp.maximum(b2 * b2 - 16 * c2, 0.0))
    r2 = (b2 + sq2) / 2
    a3 = 4 * min_overlap
    b3 = -2 * min_overlap * (height + width)
    c3 = (min_overlap - 1) * width * height
    sq3 = jnp.sqrt(jnp.maximum(b3 * b3 - 4 * a3 * c3, 0.0))
    r3 = (b3 + sq3) / 2
    return jnp.minimum(jnp.minimum(r1, r2), r3)


def _splat_one(acc_ref, rows1, cols1, cxf, syf, si, rf2, ninv):
    s = pl.multiple_of(si, 8)
    dx = cols1 - cxf
    dy = rows1 + syf
    dx2 = dx * dx
    dy2 = dy * dy
    ex = jnp.where(dx2 <= rf2, dx2 * ninv, _NEG)
    ey = jnp.where(dy2 <= rf2, dy2 * ninv, _NEG)
    acc_ref[pl.ds(s, _SLAB), :] = jnp.maximum(
        acc_ref[pl.ds(s, _SLAB), :], ey + ex)


def _tile_kernel(cx_ref, sy_ref, si_ref, r2_ref, ni_ref,
                 shm_ref, thm_ref, sreg_ref, treg_ref, out_ref, *accs):
    b = pl.program_id(0)

    for acc in accs:
        acc[...] = jnp.full_like(acc, _NEG)

    rows1 = jax.lax.broadcasted_iota(jnp.int32, (_SLAB, 1), 0).astype(jnp.float32)
    cols1 = jax.lax.broadcasted_iota(jnp.int32, (1, _W), 1).astype(jnp.float32)

    n = cx_ref.shape[1]
    nacc = len(accs)
    q = n // (nacc * 4)

    def box_body(i, carry):
        for u in range(4):
            for k, acc in enumerate(accs):
                j = (4 * i + u) + k * 4 * q
                _splat_one(acc, rows1, cols1, cx_ref[b, j], sy_ref[b, j],
                           si_ref[b, j], r2_ref[b, j], ni_ref[b, j])
        return carry

    jax.lax.fori_loop(0, q, box_body, 0)
    m = [acc[...] for acc in accs]
    while len(m) > 1:
        m = [jnp.maximum(m[2 * t], m[2 * t + 1]) for t in range(len(m) // 2)]
    macc = m[0]
    mask = jnp.where(macc >= _LOG_EPS64, jnp.exp(macc), 0.0)

    stu_max = jnp.max(shm_ref[...], axis=0)
    fuse = jnp.clip(jax.nn.sigmoid(thm_ref[...] * 0.5), 0.001, 0.999)
    fuse_max = jnp.max(fuse, axis=0)
    diff_cls = jnp.abs(stu_max - fuse_max) * mask
    reg_l1 = jnp.sum(jnp.abs(sreg_ref[...] - treg_ref[...]), axis=0)
    diff_reg = reg_l1 * (1.0 / 11.0) * mask

    out_ref[...] = jnp.concatenate(
        [jnp.sum(mask, axis=0, keepdims=True),
         jnp.sum(diff_cls, axis=0, keepdims=True),
         jnp.sum(diff_reg, axis=0, keepdims=True)], axis=0)


def _finalize_kernel(p_ref, o_ref):
    w = jnp.sum(p_ref[:, 0, :])
    c = jnp.sum(p_ref[:, 1, :])
    r = jnp.sum(p_ref[:, 2, :])
    denom = 1.0 / (w + 0.0001)
    lane = jax.lax.broadcasted_iota(jnp.int32, (1, 128), 1)
    o_ref[...] = jnp.where(lane == 0, c * denom,
                           jnp.where(lane == 1, r * denom, 0.0))


def kernel(stu_hm, tea_hm, stu_reg, tea_reg, gt_boxes):
    B, Ccls, H, W = stu_hm.shape
    Creg = stu_reg.shape[1]

    bsum = jnp.sum(gt_boxes, axis=-1)
    valid = jnp.cumprod((bsum != 0).astype(jnp.int32), axis=1).astype(bool)
    w_pix = gt_boxes[..., 3] / _VX
    h_pix = gt_boxes[..., 4] / _VY
    rad = jnp.maximum(0, _radius(w_pix, h_pix).astype(jnp.int32))
    cx = ((gt_boxes[..., 0] - _PC_X0) / _VX).astype(jnp.int32)
    cy = ((gt_boxes[..., 1] - _PC_Y0) / _VY).astype(jnp.int32)
    cxf = jnp.where(valid, cx.astype(jnp.float32), -4.0e4)
    cyf = jnp.where(valid, cy.astype(jnp.float32), -4.0e4)
    rf = jnp.where(valid, rad.astype(jnp.float32), 0.0)
    sf = jnp.clip(jnp.floor((cyf - rf) * 0.125) * 8.0, 0.0, float(_H - _SLAB))
    si = sf.astype(jnp.int32)
    syf = sf - cyf
    rf2 = rf * rf
    sigma = (2.0 * rf + 1.0) / 6.0
    ninv = -1.0 / (2.0 * sigma * sigma)

    smem = pl.BlockSpec(memory_space=pltpu.SMEM)
    partials = pl.pallas_call(
        _tile_kernel,
        grid=(B,),
        in_specs=[
            smem, smem, smem, smem, smem,
            pl.BlockSpec((None, Ccls, H, W), lambda b: (b, 0, 0, 0)),
            pl.BlockSpec((None, Ccls, H, W), lambda b: (b, 0, 0, 0)),
            pl.BlockSpec((None, Creg, H, W), lambda b: (b, 0, 0, 0)),
            pl.BlockSpec((None, Creg, H, W), lambda b: (b, 0, 0, 0)),
        ],
        out_specs=pl.BlockSpec((None, 3, W), lambda b: (b, 0, 0)),
        out_shape=jax.ShapeDtypeStruct((B, 3, W), jnp.float32),
        scratch_shapes=[pltpu.VMEM((_H, _W), jnp.float32)] * 8,
        compiler_params=pltpu.CompilerParams(
            dimension_semantics=("parallel",),
            vmem_limit_bytes=56 * 1024 * 1024),
        name="distill_tiles",
    )(cxf, syf, si, rf2, ninv, stu_hm, tea_hm, stu_reg, tea_reg)

    losses = pl.pallas_call(
        _finalize_kernel,
        out_shape=jax.ShapeDtypeStruct((1, 128), jnp.float32),
        name="distill_finalize",
    )(partials)

    return (losses[0, 0], losses[0, 1])


# speedup vs baseline: 1.1530x; 1.1530x over previous
import jax
import jax.numpy as jnp
from jax.experimental import pallas as pl
from jax.experimental.pallas import tpu as pltpu

_PC_X0 = -54.0
_PC_Y0 = -54.0
_VX = 0.075 * 4.0
_VY = 0.075 * 4.0
_LOG_EPS64 = -36.04365338911715
_NEG = -3.0e38

_H = 360
_W = 360
_SLAB = 40


def _radius(height, width, min_overlap=0.5):
    b1 = height + width
    c1 = width * height * (1 - min_overlap) / (1 + min_overlap)
    sq1 = jnp.sqrt(jnp.maximum(b1 * b1 - 4 * c1, 0.0))
    r1 = (b1 + sq1) / 2
    b2 = 2 * (height + width)
    c2 = (1 - min_overlap) * width * height
    sq2 = jnp.sqrt(jnp.maximum(b2 * b2 - 16 * c2, 0.0))
    r2 = (b2 + sq2) / 2
    a3 = 4 * min_overlap
    b3 = -2 * min_overlap * (height + width)
    c3 = (min_overlap - 1) * width * height
    sq3 = jnp.sqrt(jnp.maximum(b3 * b3 - 4 * a3 * c3, 0.0))
    r3 = (b3 + sq3) / 2
    return jnp.minimum(jnp.minimum(r1, r2), r3)


def _splat_one(acc_ref, rows1, cols1, cxf, syf, si, rf2, ninv):
    s = pl.multiple_of(si, 8)
    dx = cols1 - cxf
    dy = rows1 + syf
    dx2 = dx * dx
    dy2 = dy * dy
    ex = jnp.where(dx2 <= rf2, dx2 * ninv, _NEG)
    ey = jnp.where(dy2 <= rf2, dy2 * ninv, _NEG)
    acc_ref[pl.ds(s, _SLAB), :] = jnp.maximum(
        acc_ref[pl.ds(s, _SLAB), :], ey + ex)


def _tile_kernel(cx_ref, sy_ref, si_ref, r2_ref, ni_ref,
                 shm_ref, thm_ref, sreg_ref, treg_ref, out_ref, *accs):
    b = pl.program_id(0)

    for acc in accs:
        acc[...] = jnp.full_like(acc, _NEG)

    rows1 = jax.lax.broadcasted_iota(jnp.int32, (_SLAB, 1), 0).astype(jnp.float32)
    cols1 = jax.lax.broadcasted_iota(jnp.int32, (1, _W), 1).astype(jnp.float32)

    n = cx_ref.shape[1]
    nacc = len(accs)
    q = n // (nacc * 2)

    def box_body(i, carry):
        for u in range(2):
            for k, acc in enumerate(accs):
                j = (2 * i + u) + k * 2 * q
                _splat_one(acc, rows1, cols1, cx_ref[b, j], sy_ref[b, j],
                           si_ref[b, j], r2_ref[b, j], ni_ref[b, j])
        return carry

    jax.lax.fori_loop(0, q, box_body, 0)
    m = [acc[...] for acc in accs]
    while len(m) > 1:
        m = [jnp.maximum(m[2 * t], m[2 * t + 1]) for t in range(len(m) // 2)]
    macc = m[0]
    mask = jnp.where(macc >= _LOG_EPS64, jnp.exp(macc), 0.0)

    stu_max = jnp.max(shm_ref[...], axis=0)
    tea_max = jnp.max(thm_ref[...], axis=0)
    fuse_max = jnp.clip(jax.nn.sigmoid(tea_max * 0.5), 0.001, 0.999)
    diff_cls = jnp.abs(stu_max - fuse_max) * mask
    reg_l1 = jnp.sum(jnp.abs(sreg_ref[...] - treg_ref[...]), axis=0)
    diff_reg = reg_l1 * (1.0 / 11.0) * mask

    out_ref[...] = jnp.concatenate(
        [jnp.sum(mask, axis=0, keepdims=True),
         jnp.sum(diff_cls, axis=0, keepdims=True),
         jnp.sum(diff_reg, axis=0, keepdims=True)], axis=0)


def _finalize_kernel(p_ref, o_ref):
    w = jnp.sum(p_ref[:, 0, :])
    c = jnp.sum(p_ref[:, 1, :])
    r = jnp.sum(p_ref[:, 2, :])
    denom = 1.0 / (w + 0.0001)
    lane = jax.lax.broadcasted_iota(jnp.int32, (1, 128), 1)
    o_ref[...] = jnp.where(lane == 0, c * denom,
                           jnp.where(lane == 1, r * denom, 0.0))


def kernel(stu_hm, tea_hm, stu_reg, tea_reg, gt_boxes):
    B, Ccls, H, W = stu_hm.shape
    Creg = stu_reg.shape[1]

    bsum = jnp.sum(gt_boxes, axis=-1)
    valid = jnp.cumprod((bsum != 0).astype(jnp.int32), axis=1).astype(bool)
    w_pix = gt_boxes[..., 3] / _VX
    h_pix = gt_boxes[..., 4] / _VY
    rad = jnp.maximum(0, _radius(w_pix, h_pix).astype(jnp.int32))
    cx = ((gt_boxes[..., 0] - _PC_X0) / _VX).astype(jnp.int32)
    cy = ((gt_boxes[..., 1] - _PC_Y0) / _VY).astype(jnp.int32)
    cxf = jnp.where(valid, cx.astype(jnp.float32), -4.0e4)
    cyf = jnp.where(valid, cy.astype(jnp.float32), -4.0e4)
    rf = jnp.where(valid, rad.astype(jnp.float32), 0.0)
    sf = jnp.clip(jnp.floor((cyf - rf) * 0.125) * 8.0, 0.0, float(_H - _SLAB))
    si = sf.astype(jnp.int32)
    syf = sf - cyf
    rf2 = rf * rf
    sigma = (2.0 * rf + 1.0) / 6.0
    ninv = -1.0 / (2.0 * sigma * sigma)

    smem = pl.BlockSpec(memory_space=pltpu.SMEM)
    partials = pl.pallas_call(
        _tile_kernel,
        grid=(B,),
        in_specs=[
            smem, smem, smem, smem, smem,
            pl.BlockSpec((None, Ccls, H, W), lambda b: (b, 0, 0, 0)),
            pl.BlockSpec((None, Ccls, H, W), lambda b: (b, 0, 0, 0)),
            pl.BlockSpec((None, Creg, H, W), lambda b: (b, 0, 0, 0)),
            pl.BlockSpec((None, Creg, H, W), lambda b: (b, 0, 0, 0)),
        ],
        out_specs=pl.BlockSpec((None, 3, W), lambda b: (b, 0, 0)),
        out_shape=jax.ShapeDtypeStruct((B, 3, W), jnp.float32),
        scratch_shapes=[pltpu.VMEM((_H, _W), jnp.float32)] * 8,
        compiler_params=pltpu.CompilerParams(
            dimension_semantics=("parallel",),
            vmem_limit_bytes=56 * 1024 * 1024),
        name="distill_tiles",
    )(cxf, syf, si, rf2, ninv, stu_hm, tea_hm, stu_reg, tea_reg)

    losses = pl.pallas_call(
        _finalize_kernel,
        out_shape=jax.ShapeDtypeStruct((1, 128), jnp.float32),
        name="distill_finalize",
    )(partials)

    return (losses[0, 0], losses[0, 1])


# speedup vs baseline: 1.1677x; 1.0127x over previous
import jax
import jax.numpy as jnp
from jax.experimental import pallas as pl
from jax.experimental.pallas import tpu as pltpu

_PC_X0 = -54.0
_PC_Y0 = -54.0
_VX = 0.075 * 4.0
_VY = 0.075 * 4.0
_LOG_EPS64 = -36.04365338911715
_NEG = -3.0e38

_H = 360
_W = 360
_SLAB = 40


def _radius(height, width, min_overlap=0.5):
    b1 = height + width
    c1 = width * height * (1 - min_overlap) / (1 + min_overlap)
    sq1 = jnp.sqrt(jnp.maximum(b1 * b1 - 4 * c1, 0.0))
    r1 = (b1 + sq1) / 2
    b2 = 2 * (height + width)
    c2 = (1 - min_overlap) * width * height
    sq2 = jnp.sqrt(jnp.maximum(b2 * b2 - 16 * c2, 0.0))
    r2 = (b2 + sq2) / 2
    a3 = 4 * min_overlap
    b3 = -2 * min_overlap * (height + width)
    c3 = (min_overlap - 1) * width * height
    sq3 = jnp.sqrt(jnp.maximum(b3 * b3 - 4 * a3 * c3, 0.0))
    r3 = (b3 + sq3) / 2
    return jnp.minimum(jnp.minimum(r1, r2), r3)


def _splat_one(acc_ref, rows1, cols1, cxf, syf, si, rf2, ninv):
    s = pl.multiple_of(si, 8)
    dx = cols1 - cxf
    dy = rows1 + syf
    dx2 = dx * dx
    dy2 = dy * dy
    ex = jnp.where(dx2 <= rf2, dx2 * ninv, _NEG)
    ey = jnp.where(dy2 <= rf2, dy2 * ninv, _NEG)
    acc_ref[pl.ds(s, _SLAB), :] = jnp.maximum(
        acc_ref[pl.ds(s, _SLAB), :], ey + ex)


def _tile_kernel(cx_ref, sy_ref, si_ref, r2_ref, ni_ref,
                 shm_ref, thm_ref, sreg_ref, treg_ref, out_ref, *accs):
    b = pl.program_id(0)

    for acc in accs:
        acc[...] = jnp.full_like(acc, _NEG)

    rows1 = jax.lax.broadcasted_iota(jnp.int32, (_SLAB, 1), 0).astype(jnp.float32)
    cols1 = jax.lax.broadcasted_iota(jnp.int32, (1, _W), 1).astype(jnp.float32)

    n = cx_ref.shape[1]
    nacc = len(accs)
    q = n // (nacc * 2)

    def box_body(i, carry):
        for u in range(2):
            for k, acc in enumerate(accs):
                j = (2 * i + u) + k * 2 * q
                _splat_one(acc, rows1, cols1, cx_ref[b, j], sy_ref[b, j],
                           si_ref[b, j], r2_ref[b, j], ni_ref[b, j])
        return carry

    jax.lax.fori_loop(0, q, box_body, 0)
    m = [acc[...] for acc in accs]
    while len(m) > 1:
        m = [jnp.maximum(m[2 * t], m[2 * t + 1]) for t in range(len(m) // 2)]
    macc = m[0]
    mask = jnp.where(macc >= _LOG_EPS64, jnp.exp(macc), 0.0)

    stu_max = jnp.max(shm_ref[...], axis=0)
    tea_max = jnp.max(thm_ref[...], axis=0)
    fuse_max = jnp.clip(jax.nn.sigmoid(tea_max * 0.5), 0.001, 0.999)
    diff_cls = jnp.abs(stu_max - fuse_max) * mask
    reg_l1 = jnp.sum(jnp.abs(sreg_ref[...] - treg_ref[...]), axis=0)
    diff_reg = reg_l1 * (1.0 / 11.0) * mask

    out_ref[...] = jnp.concatenate(
        [jnp.sum(mask, axis=0, keepdims=True),
         jnp.sum(diff_cls, axis=0, keepdims=True),
         jnp.sum(diff_reg, axis=0, keepdims=True)], axis=0)


def _finalize_kernel(p_ref, o_ref):
    w = jnp.sum(p_ref[:, 0, :])
    c = jnp.sum(p_ref[:, 1, :])
    r = jnp.sum(p_ref[:, 2, :])
    denom = 1.0 / (w + 0.0001)
    lane = jax.lax.broadcasted_iota(jnp.int32, (1, 128), 1)
    o_ref[...] = jnp.where(lane == 0, c * denom,
                           jnp.where(lane == 1, r * denom, 0.0))


def kernel(stu_hm, tea_hm, stu_reg, tea_reg, gt_boxes):
    B, Ccls, H, W = stu_hm.shape
    Creg = stu_reg.shape[1]

    bsum = jnp.sum(gt_boxes, axis=-1)
    valid = jnp.cumprod((bsum != 0).astype(jnp.int32), axis=1).astype(bool)
    w_pix = gt_boxes[..., 3] / _VX
    h_pix = gt_boxes[..., 4] / _VY
    rad = jnp.maximum(0, _radius(w_pix, h_pix).astype(jnp.int32))
    cx = ((gt_boxes[..., 0] - _PC_X0) / _VX).astype(jnp.int32)
    cy = ((gt_boxes[..., 1] - _PC_Y0) / _VY).astype(jnp.int32)
    cxf = jnp.where(valid, cx.astype(jnp.float32), -4.0e4)
    cyf = jnp.where(valid, cy.astype(jnp.float32), -4.0e4)
    rf = jnp.where(valid, rad.astype(jnp.float32), 0.0)
    sf = jnp.clip(jnp.floor((cyf - rf) * 0.125) * 8.0, 0.0, float(_H - _SLAB))
    si = sf.astype(jnp.int32)
    syf = sf - cyf
    rf2 = rf * rf
    sigma = (2.0 * rf + 1.0) / 6.0
    ninv = -1.0 / (2.0 * sigma * sigma)

    smem = pl.BlockSpec(memory_space=pltpu.SMEM)
    partials = pl.pallas_call(
        _tile_kernel,
        grid=(B,),
        in_specs=[
            smem, smem, smem, smem, smem,
            pl.BlockSpec((None, Ccls, H, W), lambda b: (b, 0, 0, 0)),
            pl.BlockSpec((None, Ccls, H, W), lambda b: (b, 0, 0, 0)),
            pl.BlockSpec((None, Creg, H, W), lambda b: (b, 0, 0, 0)),
            pl.BlockSpec((None, Creg, H, W), lambda b: (b, 0, 0, 0)),
        ],
        out_specs=pl.BlockSpec((None, 3, W), lambda b: (b, 0, 0)),
        out_shape=jax.ShapeDtypeStruct((B, 3, W), jnp.float32),
        scratch_shapes=[pltpu.VMEM((_H, _W), jnp.float32)] * 4,
        compiler_params=pltpu.CompilerParams(
            dimension_semantics=("parallel",),
            vmem_limit_bytes=56 * 1024 * 1024),
        name="distill_tiles",
    )(cxf, syf, si, rf2, ninv, stu_hm, tea_hm, stu_reg, tea_reg)

    losses = pl.pallas_call(
        _finalize_kernel,
        out_shape=jax.ShapeDtypeStruct((1, 128), jnp.float32),
        name="distill_finalize",
    )(partials)

    return (losses[0, 0], losses[0, 1])
